# baseline (speedup 1.0000x reference)
import functools
import jax
import jax.numpy as jnp
from jax import lax
from jax.experimental import pallas as pl
from jax.experimental.pallas import tpu as pltpu
from jax.experimental.pallas import tpu_sc as plsc

C = 768
H = 12
DH = 64
NTOK = 676
SELN = 540
NLOW = NTOK - SELN
B = 32
LLANG = 17
NT_X = (NTOK + 1) * B
NT_XT = NTOK * B
SEQ = SELN + 4
ND2 = SEQ * B
DUMMY = ND2
NSC = 24576
ROWS_W = NSC // 32
CH = 64
NCH = ROWS_W // CH

_NC = 2
_NS = 16


def _score_body(x_ref, lang_ref, o_ref):
    ls = jnp.sum(lang_ref[...], axis=0)
    s_full = lax.dot_general(x_ref[...], ls, (((1,), (1,)), ((), ())),
                             preferred_element_type=jnp.float32)
    s3 = s_full.reshape(s_full.shape[0] // B, B, B) * (1.0 / LLANG)
    r = lax.broadcasted_iota(jnp.int32, (B, B), 0)
    c = lax.broadcasted_iota(jnp.int32, (B, B), 1)
    eye = (r == c).astype(jnp.float32)
    d = jnp.sum(s3 * eye[None], axis=2)
    o_ref[...] = d.reshape(1, *d.shape)


def _score(xt_flat, lang):
    blk = 1664
    grid = NT_XT // blk
    return pl.pallas_call(
        _score_body,
        grid=(grid,),
        in_specs=[
            pl.BlockSpec((blk, C), lambda i: (i, 0)),
            pl.BlockSpec((LLANG, B, C), lambda i: (0, 0, 0)),
        ],
        out_specs=pl.BlockSpec((1, blk // B, B), lambda i: (i, 0, 0)),
        out_shape=jax.ShapeDtypeStruct((grid, blk // B, B), jnp.float32),
    )(xt_flat, lang).reshape(NTOK, B)


def _select_body(s_ref, dst_ref, src_ref, g_ref, w_ref):
    s = s_ref[...].T + 0.0
    nio = lax.broadcasted_iota(jnp.int32, (B, NTOK), 1)
    valid = (nio < NTOK).astype(jnp.int32)

    u = lax.bitcast_convert_type(s, jnp.int32)
    ki = jnp.where(u >= 0, u, u ^ jnp.int32(0x7FFFFFFF))
    ku = ki ^ jnp.int32(-2147483648)

    def step(k, carry):
        gt, eq, need = carry
        bit = 31 - k
        bv = lax.shift_right_logical(ku, bit) & 1
        cnt = jnp.sum(eq * bv, axis=1, keepdims=True)
        cond = cnt >= need
        gt = gt | jnp.where(cond, 0, eq * bv)
        eq = jnp.where(cond, eq * bv, eq * (1 - bv))
        need = need - jnp.where(cond, 0, cnt)
        return gt, eq, need

    gt0 = jnp.zeros((B, NTOK), jnp.int32)
    need0 = jnp.full((B, 1), SELN, jnp.int32)
    gt, eq, need = lax.fori_loop(0, 32, step, (gt0, valid, need0))

    rr = lax.broadcasted_iota(jnp.int32, (NTOK, NTOK), 0)
    cc = lax.broadcasted_iota(jnp.int32, (NTOK, NTOK), 1)
    tmat = (rr < cc).astype(jnp.float32)
    eqf = eq.astype(jnp.float32)
    eqrank = lax.dot_general(eqf, tmat, (((1,), (0,)), ((), ())),
                             preferred_element_type=jnp.float32)
    high = gt | (eq * (eqrank.astype(jnp.int32) < need).astype(jnp.int32))
    low = valid * (1 - high)

    def flat_prefix(mask_i32):
        mf = mask_i32.astype(jnp.float32)
        rowcnt = jnp.sum(mf, axis=0, keepdims=True)
        rowpref = lax.dot_general(rowcnt, tmat, (((1,), (0,)), ((), ())),
                                  preferred_element_type=jnp.float32)
        bb1 = lax.broadcasted_iota(jnp.int32, (B, B), 0)
        bb2 = lax.broadcasted_iota(jnp.int32, (B, B), 1)
        tb = (bb1 < bb2).astype(jnp.float32)
        colcnt = lax.dot_general(tb, mf, (((0,), (0,)), ((), ())),
                                 preferred_element_type=jnp.float32)
        return (rowpref + colcnt).astype(jnp.int32)

    pos_hi = flat_prefix(high)
    pos_lo = flat_prefix(low)

    c_lo = pos_lo & (B - 1)
    lowb = low == 1
    mfull = jnp.zeros((B, NTOK), jnp.float32)
    for j in range(B):
        sel = lowb & (c_lo == j)
        mj = jnp.max(jnp.where(sel, s, -jnp.inf))
        mfull = jnp.where(c_lo == j, mj, mfull)
    e = jnp.where(lowb, jnp.exp(s - mfull), 0.0)
    dfull = jnp.ones((B, NTOK), jnp.float32)
    for j in range(B):
        dj = jnp.sum(jnp.where(c_lo == j, e, 0.0))
        dfull = jnp.where(c_lo == j, dj, dfull)
    w = e / dfull

    hi_row = (pos_hi & (B - 1)) * SEQ + 1 + lax.shift_right_logical(pos_hi, 5)
    biota = lax.broadcasted_iota(jnp.int32, (1, B), 1)
    wk = lax.shift_right_logical((nio + 1) * 2731, 16)
    dstx = jnp.where(high == 1, hi_row, DUMMY + wk).T
    dst_ref[...] = jnp.concatenate([biota * SEQ, dstx], axis=0)
    lo_div = lax.shift_right_logical(pos_lo * 7711, 20)
    srcx = jnp.where(high == 1, hi_row, pos_lo * 4).T
    src_ref[...] = jnp.concatenate([biota * SEQ, srcx], axis=0)
    gx = jnp.where(high == 1, B, lo_div).T
    g_ref[...] = jnp.concatenate([jnp.full((1, B), B, jnp.int32), gx], axis=0)

    jio = lax.broadcasted_iota(jnp.int32, (NTOK, B, B), 2)
    w_ref[...] = w.T[:, :, None] * (c_lo.T[:, :, None] == jio).astype(jnp.float32)


def _select(score):
    return pl.pallas_call(
        _select_body,
        out_shape=(
            jax.ShapeDtypeStruct((NTOK + 1, B), jnp.int32),
            jax.ShapeDtypeStruct((NTOK + 1, B), jnp.int32),
            jax.ShapeDtypeStruct((NTOK + 1, B), jnp.int32),
            jax.ShapeDtypeStruct((NTOK, B, B), jnp.float32),
        ),
    )(score)


def _lowagg_body(w_ref, x_ref, o_ref):
    part = lax.dot_general(w_ref[...], x_ref[...], (((0,), (0,)), ((), ())),
                           preferred_element_type=jnp.float32)

    @pl.when(pl.program_id(0) == 0)
    def _():
        o_ref[...] = jnp.zeros_like(o_ref)

    o_ref[...] += part


def _lowagg(w_flat, xt_flat):
    blk = 1664
    grid = NT_XT // blk
    return pl.pallas_call(
        _lowagg_body,
        grid=(grid,),
        in_specs=[
            pl.BlockSpec((blk, B), lambda i: (i, 0)),
            pl.BlockSpec((blk, C), lambda i: (i, 0)),
        ],
        out_specs=pl.BlockSpec((B, C), lambda i: (0, 0)),
        out_shape=jax.ShapeDtypeStruct((B, C), jnp.float32),
    )(w_flat, xt_flat)


def _sc_scatter_rows(x_sc, dst3):
    mesh = plsc.VectorSubcoreMesh(core_axis_name="c", subcore_axis_name="s")

    @functools.partial(
        pl.kernel,
        out_type=jax.ShapeDtypeStruct((ND2 + 40, C), jnp.float32),
        mesh=mesh,
        scratch_types=[
            pltpu.VMEM((NCH, CH), jnp.int32),
            pltpu.VMEM((CH, C), jnp.float32),
            pltpu.VMEM((CH, C), jnp.float32),
            pltpu.SemaphoreType.DMA,
            pltpu.SemaphoreType.DMA,
            pltpu.SemaphoreType.DMA,
            pltpu.SemaphoreType.DMA,
        ],
    )
    def k(x_hbm, dst_hbm, out_hbm, idx_v, row0, row1, ss0, ss1, cs0, cs1):
        wid = lax.axis_index("s") * _NC + lax.axis_index("c")
        base = wid * ROWS_W
        pltpu.sync_copy(dst_hbm.at[wid], idx_v)
        rows, ssem, csem = (row0, row1), (ss0, ss1), (cs0, cs1)

        def stage(ch, p):
            return pltpu.async_copy(
                x_hbm.at[pl.ds(base + ch * CH, CH)], rows[p], ssem[p])

        st = {0: stage(0, 0)}
        scat = {}
        for ch in range(NCH):
            p = ch & 1
            st[ch].wait()
            if ch + 1 < NCH:
                if ch - 1 in scat:
                    scat[ch - 1].wait()
                st[ch + 1] = stage(ch + 1, 1 - p)
            scat[ch] = pltpu.async_copy(rows[p], out_hbm.at[idx_v.at[ch]], csem[p])
        scat[NCH - 2].wait()
        scat[NCH - 1].wait()

    return k(x_sc, dst3)


def _sc_gather_rows(delta_out, src3):
    mesh = plsc.VectorSubcoreMesh(core_axis_name="c", subcore_axis_name="s")

    @functools.partial(
        pl.kernel,
        out_type=jax.ShapeDtypeStruct((NSC, C), jnp.float32),
        mesh=mesh,
        scratch_types=[
            pltpu.VMEM((NCH, CH), jnp.int32),
            pltpu.VMEM((CH, C), jnp.float32),
            pltpu.VMEM((CH, C), jnp.float32),
            pltpu.SemaphoreType.DMA,
            pltpu.SemaphoreType.DMA,
            pltpu.SemaphoreType.DMA,
            pltpu.SemaphoreType.DMA,
        ],
    )
    def k(d_hbm, src_hbm, out_hbm, idx_v, row0, row1, gs0, gs1, ws0, ws1):
        wid = lax.axis_index("s") * _NC + lax.axis_index("c")
        base = wid * ROWS_W
        pltpu.sync_copy(src_hbm.at[wid], idx_v)
        rows, gsem, wsem = (row0, row1), (gs0, gs1), (ws0, ws1)

        def gath(ch, p):
            return pltpu.async_copy(d_hbm.at[idx_v.at[ch]], rows[p], gsem[p])

        g = {0: gath(0, 0)}
        wr = {}
        for ch in range(NCH):
            p = ch & 1
            g[ch].wait()
            if ch + 1 < NCH:
                if ch - 1 in wr:
                    wr[ch - 1].wait()
                g[ch + 1] = gath(ch + 1, 1 - p)
            wr[ch] = pltpu.async_copy(
                rows[p], out_hbm.at[pl.ds(base + ch * CH, CH)], wsem[p])
        wr[NCH - 2].wait()
        wr[NCH - 1].wait()

    return k(delta_out, src3)


def _mmln_body(x_ref, w_ref, b_ref, lw_ref, lb_ref, o_ref):
    xb = x_ref[...]
    m = jnp.mean(xb, axis=1, keepdims=True)
    xc = xb - m
    v = jnp.mean(xc * xc, axis=1, keepdims=True)
    ln = xc * lax.rsqrt(v + 1e-5) * lw_ref[...] + lb_ref[...]
    r = lax.dot_general(ln, w_ref[...], (((1,), (1,)), ((), ())),
                        preferred_element_type=jnp.float32) + b_ref[...]
    o_ref[...] = r.astype(o_ref.dtype)


def _mm_body(x_ref, w_ref, b_ref, o_ref):
    r = lax.dot_general(x_ref[...], w_ref[...], (((1,), (1,)), ((), ())),
                        preferred_element_type=jnp.float32) + b_ref[...]
    o_ref[...] = r.astype(o_ref.dtype)


def _matmul(x, w, bias, ln_wb=None, blk=512, out_dtype=jnp.float32):
    n, kdim = x.shape
    nout = w.shape[0]
    grid = n // blk
    if ln_wb is None:
        body, extra = _mm_body, []
    else:
        body, extra = _mmln_body, [ln_wb[0].reshape(1, kdim), ln_wb[1].reshape(1, kdim)]
    in_specs = [pl.BlockSpec((blk, kdim), lambda i: (i, 0)),
                pl.BlockSpec((nout, kdim), lambda i: (0, 0)),
                pl.BlockSpec((1, nout), lambda i: (0, 0))]
    in_specs += [pl.BlockSpec((1, kdim), lambda i: (0, 0))] * len(extra)
    return pl.pallas_call(
        body,
        grid=(grid,),
        in_specs=in_specs,
        out_specs=pl.BlockSpec((blk, nout), lambda i: (i, 0)),
        out_shape=jax.ShapeDtypeStruct((n, nout), out_dtype),
    )(x, w, bias.reshape(1, nout), *extra)


def _attn_body(qkv_ref, o_ref):
    blk = qkv_ref[...]
    kmask = lax.broadcasted_iota(jnp.int32, (SEQ, SEQ), 1) < SELN + 2
    parts = []
    for h in range(H):
        qh = blk[:, h * DH:(h + 1) * DH]
        kh = blk[:, C + h * DH:C + (h + 1) * DH]
        vh = blk[:, 2 * C + h * DH:2 * C + (h + 1) * DH]
        sc = lax.dot_general(qh, kh, (((1,), (1,)), ((), ())),
                             preferred_element_type=jnp.float32) * 0.125
        sc = jnp.where(kmask, sc, -1e30)
        m = jnp.max(sc, axis=1, keepdims=True)
        p = jnp.exp(sc - m)
        p = p / jnp.sum(p, axis=1, keepdims=True)
        parts.append(lax.dot_general(p, vh, (((1,), (0,)), ((), ())),
                                     preferred_element_type=jnp.float32))
    o_ref[...] = jnp.concatenate(parts, axis=1)


def _attention(qkv):
    return pl.pallas_call(
        _attn_body,
        grid=(B,),
        in_specs=[pl.BlockSpec((SEQ, 3 * C), lambda b: (b, 0))],
        out_specs=pl.BlockSpec((SEQ, C), lambda b: (b, 0)),
        out_shape=jax.ShapeDtypeStruct((ND2, C), jnp.float32),
    )(qkv)


def _mlp_body(x_ref, t_ref, g_ref, lo_ref, fw_ref, fb_ref, pw_ref, pb_ref,
              lw_ref, lb_ref, o_ref):
    g = g_ref[...]
    oh = (g == lax.broadcasted_iota(jnp.int32, (g.shape[0], B), 1)).astype(
        jnp.float32)
    lowadd = lax.dot_general(oh, lo_ref[...], (((1,), (0,)), ((), ())),
                             preferred_element_type=jnp.float32)
    t_eff = jnp.where(g < B, lowadd, t_ref[...])
    x2 = x_ref[...] + t_eff
    m = jnp.mean(x2, axis=1, keepdims=True)
    xc = x2 - m
    v = jnp.mean(xc * xc, axis=1, keepdims=True)
    ln = xc * lax.rsqrt(v + 1e-5) * lw_ref[...] + lb_ref[...]
    h1 = lax.dot_general(ln, fw_ref[...], (((1,), (1,)), ((), ())),
                         preferred_element_type=jnp.float32) + fb_ref[...]
    g = h1 / (1.0 + jnp.exp(-1.702 * h1))
    h2 = lax.dot_general(g, pw_ref[...], (((1,), (1,)), ((), ())),
                         preferred_element_type=jnp.float32) + pb_ref[...]
    o_ref[...] = x2 + h2


def _mlp(x_pad, temple, g_idx, lowout, fc_w, fc_b, proj_w, proj_b, ln_w, ln_b):
    blk = 512
    grid = NSC // blk
    return pl.pallas_call(
        _mlp_body,
        grid=(grid,),
        in_specs=[
            pl.BlockSpec((blk, C), lambda i: (i, 0)),
            pl.BlockSpec((blk, C), lambda i: (i, 0)),
            pl.BlockSpec((blk, 1), lambda i: (i, 0)),
            pl.BlockSpec((B, C), lambda i: (0, 0)),
            pl.BlockSpec((4 * C, C), lambda i: (0, 0)),
            pl.BlockSpec((1, 4 * C), lambda i: (0, 0)),
            pl.BlockSpec((C, 4 * C), lambda i: (0, 0)),
            pl.BlockSpec((1, C), lambda i: (0, 0)),
            pl.BlockSpec((1, C), lambda i: (0, 0)),
            pl.BlockSpec((1, C), lambda i: (0, 0)),
        ],
        out_specs=pl.BlockSpec((blk, C), lambda i: (i, 0)),
        out_shape=jax.ShapeDtypeStruct((NSC, C), jnp.float32),
    )(x_pad, temple, g_idx, lowout, fc_w, fc_b.reshape(1, 4 * C), proj_w,
      proj_b.reshape(1, C), ln_w.reshape(1, C), ln_b.reshape(1, C))


def kernel(x, lang_tokens, ln1_w, ln1_b, in_proj_w, in_proj_b, out_proj_w,
           out_proj_b, ln2_w, ln2_b, fc_w, fc_b, proj_w, proj_b):
    x_flat = x.reshape(NT_X, C)
    xt_flat = x_flat[B:]

    score = _score(xt_flat, lang_tokens)
    dst, src, gidx, w3 = _select(score)
    low_x = _lowagg(w3.reshape(NT_XT, B), xt_flat)

    x_sc = jnp.concatenate(
        [x_flat, low_x, jnp.zeros((NSC - NT_X - B, C), jnp.float32)], axis=0)
    jcol = jnp.arange(B, dtype=jnp.int32) * SEQ
    dst_full = jnp.concatenate([
        dst.reshape(NT_X),
        jcol + (SELN + 1),
        jcol + (SELN + 2),
        jcol + (SELN + 3),
        DUMMY + jnp.arange(NT_X + 3 * B, NSC, dtype=jnp.int32) // ROWS_W,
    ])
    src_full = jnp.concatenate(
        [src.reshape(NT_X), jnp.zeros((NSC - NT_X,), jnp.int32)])

    sc_out = _sc_scatter_rows(x_sc, dst_full.reshape(B, NCH, CH))
    delta = sc_out[:ND2]

    qkv = _matmul(delta, in_proj_w, in_proj_b, ln_wb=(ln1_w, ln1_b))
    o_flat = _attention(qkv)
    delta_out = _matmul(o_flat, out_proj_w, out_proj_b)

    temple = _sc_gather_rows(delta_out, src_full.reshape(B, NCH, CH))
    g_pad = jnp.concatenate(
        [gidx.reshape(NT_X), jnp.full((NSC - NT_X,), B, jnp.int32)]
    ).reshape(NSC, 1)
    lowout = delta_out.reshape(B, SEQ, C)[:, SELN + 1, :]
    out = _mlp(x_sc, temple, g_pad, lowout, fc_w, fc_b, proj_w, proj_b,
               ln2_w, ln2_b)
    return out[:NT_X].reshape(NTOK + 1, B, C)


# speedup vs baseline: 1.0232x; 1.0232x over previous
import functools
import jax
import jax.numpy as jnp
from jax import lax
from jax.experimental import pallas as pl
from jax.experimental.pallas import tpu as pltpu
from jax.experimental.pallas import tpu_sc as plsc

C = 768
H = 12
DH = 64
NTOK = 676
SELN = 540
NLOW = NTOK - SELN
B = 32
LLANG = 17
NT_X = (NTOK + 1) * B
NT_XT = NTOK * B
SEQ = SELN + 4
ND2 = SEQ * B
DUMMY = ND2
NSC = 24576
ROWS_W = NSC // 32
CH = 64
NCH = ROWS_W // CH

_NC = 2
_NS = 16


def _score_body(x_ref, lang_ref, o_ref):
    ls = jnp.sum(lang_ref[...], axis=0)
    s_full = lax.dot_general(x_ref[...], ls, (((1,), (1,)), ((), ())),
                             preferred_element_type=jnp.float32)
    s3 = s_full.reshape(s_full.shape[0] // B, B, B) * (1.0 / LLANG)
    r = lax.broadcasted_iota(jnp.int32, (B, B), 0)
    c = lax.broadcasted_iota(jnp.int32, (B, B), 1)
    eye = (r == c).astype(jnp.float32)
    d = jnp.sum(s3 * eye[None], axis=2)
    o_ref[...] = d.reshape(1, *d.shape)


def _score(xt_flat, lang):
    blk = 1664
    grid = NT_XT // blk
    return pl.pallas_call(
        _score_body,
        grid=(grid,),
        in_specs=[
            pl.BlockSpec((blk, C), lambda i: (i, 0)),
            pl.BlockSpec((LLANG, B, C), lambda i: (0, 0, 0)),
        ],
        out_specs=pl.BlockSpec((1, blk // B, B), lambda i: (i, 0, 0)),
        out_shape=jax.ShapeDtypeStruct((grid, blk // B, B), jnp.float32),
    )(xt_flat, lang).reshape(NTOK, B)


def _select_body(s_ref, dst_ref, src_ref, w_ref):
    s = s_ref[...].T + 0.0
    nio = lax.broadcasted_iota(jnp.int32, (B, NTOK), 1)
    valid = (nio < NTOK).astype(jnp.int32)

    u = lax.bitcast_convert_type(s, jnp.int32)
    ki = jnp.where(u >= 0, u, u ^ jnp.int32(0x7FFFFFFF))
    ku = ki ^ jnp.int32(-2147483648)

    def step(k, carry):
        gt, eq, need = carry
        bit = 31 - k
        bv = lax.shift_right_logical(ku, bit) & 1
        cnt = jnp.sum(eq * bv, axis=1, keepdims=True)
        cond = cnt >= need
        gt = gt | jnp.where(cond, 0, eq * bv)
        eq = jnp.where(cond, eq * bv, eq * (1 - bv))
        need = need - jnp.where(cond, 0, cnt)
        return gt, eq, need

    gt0 = jnp.zeros((B, NTOK), jnp.int32)
    need0 = jnp.full((B, 1), SELN, jnp.int32)
    gt, eq, need = lax.fori_loop(0, 32, step, (gt0, valid, need0))

    rr = lax.broadcasted_iota(jnp.int32, (NTOK, NTOK), 0)
    cc = lax.broadcasted_iota(jnp.int32, (NTOK, NTOK), 1)
    tmat = (rr < cc).astype(jnp.float32)
    eqf = eq.astype(jnp.float32)
    eqrank = lax.dot_general(eqf, tmat, (((1,), (0,)), ((), ())),
                             preferred_element_type=jnp.float32)
    high = gt | (eq * (eqrank.astype(jnp.int32) < need).astype(jnp.int32))
    low = valid * (1 - high)

    def flat_prefix(mask_i32):
        mf = mask_i32.astype(jnp.float32)
        rowcnt = jnp.sum(mf, axis=0, keepdims=True)
        rowpref = lax.dot_general(rowcnt, tmat, (((1,), (0,)), ((), ())),
                                  preferred_element_type=jnp.float32)
        bb1 = lax.broadcasted_iota(jnp.int32, (B, B), 0)
        bb2 = lax.broadcasted_iota(jnp.int32, (B, B), 1)
        tb = (bb1 < bb2).astype(jnp.float32)
        colcnt = lax.dot_general(tb, mf, (((0,), (0,)), ((), ())),
                                 preferred_element_type=jnp.float32)
        return (rowpref + colcnt).astype(jnp.int32)

    pos_hi = flat_prefix(high)
    pos_lo = flat_prefix(low)

    c_lo = pos_lo & (B - 1)
    lowb = low == 1
    mfull = jnp.zeros((B, NTOK), jnp.float32)
    for j in range(B):
        sel = lowb & (c_lo == j)
        mj = jnp.max(jnp.where(sel, s, -jnp.inf))
        mfull = jnp.where(c_lo == j, mj, mfull)
    e = jnp.where(lowb, jnp.exp(s - mfull), 0.0)
    dfull = jnp.ones((B, NTOK), jnp.float32)
    for j in range(B):
        dj = jnp.sum(jnp.where(c_lo == j, e, 0.0))
        dfull = jnp.where(c_lo == j, dj, dfull)
    w = e / dfull

    hi_row = (pos_hi & (B - 1)) * SEQ + 1 + lax.shift_right_logical(pos_hi, 5)
    biota = lax.broadcasted_iota(jnp.int32, (1, B), 1)
    wk = lax.shift_right_logical((nio + 1) * 2731, 16)
    dstx = jnp.where(high == 1, hi_row, DUMMY + wk).T
    dst_ref[...] = jnp.concatenate([biota * SEQ, dstx], axis=0)
    lo_div = lax.shift_right_logical(pos_lo * 7711, 20)
    srcx = jnp.where(high == 1, hi_row, lo_div * SEQ + SELN + 1).T
    src_ref[...] = jnp.concatenate([biota * SEQ, srcx], axis=0)

    jio = lax.broadcasted_iota(jnp.int32, (NTOK, B, B), 2)
    w_ref[...] = w.T[:, :, None] * (c_lo.T[:, :, None] == jio).astype(jnp.float32)


def _select(score):
    return pl.pallas_call(
        _select_body,
        out_shape=(
            jax.ShapeDtypeStruct((NTOK + 1, B), jnp.int32),
            jax.ShapeDtypeStruct((NTOK + 1, B), jnp.int32),
            jax.ShapeDtypeStruct((NTOK, B, B), jnp.float32),
        ),
    )(score)


def _lowagg_body(w_ref, x_ref, o_ref):
    part = lax.dot_general(w_ref[...], x_ref[...], (((0,), (0,)), ((), ())),
                           preferred_element_type=jnp.float32)

    @pl.when(pl.program_id(0) == 0)
    def _():
        o_ref[...] = jnp.zeros_like(o_ref)

    o_ref[...] += part


def _lowagg(w_flat, xt_flat):
    blk = 1664
    grid = NT_XT // blk
    return pl.pallas_call(
        _lowagg_body,
        grid=(grid,),
        in_specs=[
            pl.BlockSpec((blk, B), lambda i: (i, 0)),
            pl.BlockSpec((blk, C), lambda i: (i, 0)),
        ],
        out_specs=pl.BlockSpec((B, C), lambda i: (0, 0)),
        out_shape=jax.ShapeDtypeStruct((B, C), jnp.float32),
    )(w_flat, xt_flat)


def _sc_scatter_rows(x_sc, dst3):
    mesh = plsc.VectorSubcoreMesh(core_axis_name="c", subcore_axis_name="s")

    @functools.partial(
        pl.kernel,
        out_type=jax.ShapeDtypeStruct((ND2 + 40, C), jnp.float32),
        mesh=mesh,
        scratch_types=[
            pltpu.VMEM((NCH, CH), jnp.int32),
            pltpu.VMEM((CH, C), jnp.float32),
            pltpu.VMEM((CH, C), jnp.float32),
            pltpu.SemaphoreType.DMA,
            pltpu.SemaphoreType.DMA,
            pltpu.SemaphoreType.DMA,
            pltpu.SemaphoreType.DMA,
        ],
    )
    def k(x_hbm, dst_hbm, out_hbm, idx_v, row0, row1, ss0, ss1, cs0, cs1):
        wid = lax.axis_index("s") * _NC + lax.axis_index("c")
        base = wid * ROWS_W
        pltpu.sync_copy(dst_hbm.at[wid], idx_v)
        rows, ssem, csem = (row0, row1), (ss0, ss1), (cs0, cs1)

        def stage(ch, p):
            return pltpu.async_copy(
                x_hbm.at[pl.ds(base + ch * CH, CH)], rows[p], ssem[p])

        st = {0: stage(0, 0)}
        scat = {}
        for ch in range(NCH):
            p = ch & 1
            st[ch].wait()
            if ch + 1 < NCH:
                if ch - 1 in scat:
                    scat[ch - 1].wait()
                st[ch + 1] = stage(ch + 1, 1 - p)
            scat[ch] = pltpu.async_copy(rows[p], out_hbm.at[idx_v.at[ch]], csem[p])
        scat[NCH - 2].wait()
        scat[NCH - 1].wait()

    return k(x_sc, dst3)


def _sc_gather_rows(delta_out, src3):
    mesh = plsc.VectorSubcoreMesh(core_axis_name="c", subcore_axis_name="s")

    @functools.partial(
        pl.kernel,
        out_type=jax.ShapeDtypeStruct((NSC, C), jnp.float32),
        mesh=mesh,
        scratch_types=[
            pltpu.VMEM((NCH, CH), jnp.int32),
            pltpu.VMEM((CH, C), jnp.float32),
            pltpu.VMEM((CH, C), jnp.float32),
            pltpu.SemaphoreType.DMA,
            pltpu.SemaphoreType.DMA,
            pltpu.SemaphoreType.DMA,
            pltpu.SemaphoreType.DMA,
        ],
    )
    def k(d_hbm, src_hbm, out_hbm, idx_v, row0, row1, gs0, gs1, ws0, ws1):
        wid = lax.axis_index("s") * _NC + lax.axis_index("c")
        base = wid * ROWS_W
        pltpu.sync_copy(src_hbm.at[wid], idx_v)
        rows, gsem, wsem = (row0, row1), (gs0, gs1), (ws0, ws1)

        def gath(ch, p):
            return pltpu.async_copy(d_hbm.at[idx_v.at[ch]], rows[p], gsem[p])

        g = {0: gath(0, 0)}
        wr = {}
        for ch in range(NCH):
            p = ch & 1
            g[ch].wait()
            if ch + 1 < NCH:
                if ch - 1 in wr:
                    wr[ch - 1].wait()
                g[ch + 1] = gath(ch + 1, 1 - p)
            wr[ch] = pltpu.async_copy(
                rows[p], out_hbm.at[pl.ds(base + ch * CH, CH)], wsem[p])
        wr[NCH - 2].wait()
        wr[NCH - 1].wait()

    return k(delta_out, src3)


def _mmln_body(x_ref, w_ref, b_ref, lw_ref, lb_ref, o_ref):
    xb = x_ref[...]
    m = jnp.mean(xb, axis=1, keepdims=True)
    xc = xb - m
    v = jnp.mean(xc * xc, axis=1, keepdims=True)
    ln = xc * lax.rsqrt(v + 1e-5) * lw_ref[...] + lb_ref[...]
    r = lax.dot_general(ln, w_ref[...], (((1,), (1,)), ((), ())),
                        preferred_element_type=jnp.float32) + b_ref[...]
    o_ref[...] = r.astype(o_ref.dtype)


def _mm_body(x_ref, w_ref, b_ref, o_ref):
    r = lax.dot_general(x_ref[...], w_ref[...], (((1,), (1,)), ((), ())),
                        preferred_element_type=jnp.float32) + b_ref[...]
    o_ref[...] = r.astype(o_ref.dtype)


def _matmul(x, w, bias, ln_wb=None, blk=512, out_dtype=jnp.float32):
    n, kdim = x.shape
    nout = w.shape[0]
    grid = n // blk
    if ln_wb is None:
        body, extra = _mm_body, []
    else:
        body, extra = _mmln_body, [ln_wb[0].reshape(1, kdim), ln_wb[1].reshape(1, kdim)]
    in_specs = [pl.BlockSpec((blk, kdim), lambda i: (i, 0)),
                pl.BlockSpec((nout, kdim), lambda i: (0, 0)),
                pl.BlockSpec((1, nout), lambda i: (0, 0))]
    in_specs += [pl.BlockSpec((1, kdim), lambda i: (0, 0))] * len(extra)
    return pl.pallas_call(
        body,
        grid=(grid,),
        in_specs=in_specs,
        out_specs=pl.BlockSpec((blk, nout), lambda i: (i, 0)),
        out_shape=jax.ShapeDtypeStruct((n, nout), out_dtype),
    )(x, w, bias.reshape(1, nout), *extra)


def _attn_body(qkv_ref, o_ref):
    blk = qkv_ref[...]
    kmask = lax.broadcasted_iota(jnp.int32, (SEQ, SEQ), 1) < SELN + 2
    parts = []
    for h in range(H):
        qh = blk[:, h * DH:(h + 1) * DH]
        kh = blk[:, C + h * DH:C + (h + 1) * DH]
        vh = blk[:, 2 * C + h * DH:2 * C + (h + 1) * DH]
        sc = lax.dot_general(qh, kh, (((1,), (1,)), ((), ())),
                             preferred_element_type=jnp.float32) * 0.125
        sc = jnp.where(kmask, sc, -1e30)
        m = jnp.max(sc, axis=1, keepdims=True)
        p = jnp.exp(sc - m)
        p = p / jnp.sum(p, axis=1, keepdims=True)
        parts.append(lax.dot_general(p, vh, (((1,), (0,)), ((), ())),
                                     preferred_element_type=jnp.float32))
    o_ref[...] = jnp.concatenate(parts, axis=1)


def _attention(qkv):
    return pl.pallas_call(
        _attn_body,
        grid=(B,),
        in_specs=[pl.BlockSpec((SEQ, 3 * C), lambda b: (b, 0))],
        out_specs=pl.BlockSpec((SEQ, C), lambda b: (b, 0)),
        out_shape=jax.ShapeDtypeStruct((ND2, C), jnp.float32),
    )(qkv)


def _mlp_body(x_ref, t_ref, fw_ref, fb_ref, pw_ref, pb_ref, lw_ref, lb_ref, o_ref):
    x2 = x_ref[...] + t_ref[...]
    m = jnp.mean(x2, axis=1, keepdims=True)
    xc = x2 - m
    v = jnp.mean(xc * xc, axis=1, keepdims=True)
    ln = xc * lax.rsqrt(v + 1e-5) * lw_ref[...] + lb_ref[...]
    h1 = lax.dot_general(ln, fw_ref[...], (((1,), (1,)), ((), ())),
                         preferred_element_type=jnp.float32) + fb_ref[...]
    g = h1 / (1.0 + jnp.exp(-1.702 * h1))
    h2 = lax.dot_general(g, pw_ref[...], (((1,), (1,)), ((), ())),
                         preferred_element_type=jnp.float32) + pb_ref[...]
    o_ref[...] = x2 + h2


def _mlp(x_pad, temple, fc_w, fc_b, proj_w, proj_b, ln_w, ln_b):
    blk = 512
    grid = NSC // blk
    return pl.pallas_call(
        _mlp_body,
        grid=(grid,),
        in_specs=[
            pl.BlockSpec((blk, C), lambda i: (i, 0)),
            pl.BlockSpec((blk, C), lambda i: (i, 0)),
            pl.BlockSpec((4 * C, C), lambda i: (0, 0)),
            pl.BlockSpec((1, 4 * C), lambda i: (0, 0)),
            pl.BlockSpec((C, 4 * C), lambda i: (0, 0)),
            pl.BlockSpec((1, C), lambda i: (0, 0)),
            pl.BlockSpec((1, C), lambda i: (0, 0)),
            pl.BlockSpec((1, C), lambda i: (0, 0)),
        ],
        out_specs=pl.BlockSpec((blk, C), lambda i: (i, 0)),
        out_shape=jax.ShapeDtypeStruct((NSC, C), jnp.float32),
    )(x_pad, temple, fc_w, fc_b.reshape(1, 4 * C), proj_w,
      proj_b.reshape(1, C), ln_w.reshape(1, C), ln_b.reshape(1, C))


def kernel(x, lang_tokens, ln1_w, ln1_b, in_proj_w, in_proj_b, out_proj_w,
           out_proj_b, ln2_w, ln2_b, fc_w, fc_b, proj_w, proj_b):
    x_flat = x.reshape(NT_X, C)
    xt_flat = x_flat[B:]

    score = _score(xt_flat, lang_tokens)
    dst, src, w3 = _select(score)
    low_x = _lowagg(w3.reshape(NT_XT, B), xt_flat)

    x_sc = jnp.concatenate(
        [x_flat, low_x, jnp.zeros((NSC - NT_X - B, C), jnp.float32)], axis=0)
    jcol = jnp.arange(B, dtype=jnp.int32) * SEQ
    dst_full = jnp.concatenate([
        dst.reshape(NT_X),
        jcol + (SELN + 1),
        jcol + (SELN + 2),
        jcol + (SELN + 3),
        DUMMY + jnp.arange(NT_X + 3 * B, NSC, dtype=jnp.int32) // ROWS_W,
    ])
    src_full = jnp.concatenate(
        [src.reshape(NT_X), jnp.zeros((NSC - NT_X,), jnp.int32)])

    sc_out = _sc_scatter_rows(x_sc, dst_full.reshape(B, NCH, CH))
    delta = sc_out[:ND2]

    qkv = _matmul(delta, in_proj_w, in_proj_b, ln_wb=(ln1_w, ln1_b))
    o_flat = _attention(qkv)
    delta_out = _matmul(o_flat, out_proj_w, out_proj_b)

    temple = _sc_gather_rows(delta_out, src_full.reshape(B, NCH, CH))
    out = _mlp(x_sc, temple, fc_w, fc_b, proj_w, proj_b, ln2_w, ln2_b)
    return out[:NT_X].reshape(NTOK + 1, B, C)
